# baseline (speedup 1.0000x reference)
import functools

import numpy as np
import jax
import jax.numpy as jnp
from jax import lax
from jax.experimental import pallas as pl
from jax.experimental.pallas import tpu as pltpu
from jax.experimental.pallas import tpu_sc as plsc

_N = 10000
_N2 = _N // 2
_E = 320000
_DF = 128
_DS = 58
_H = 64
_ALPHA = 0.4
_THETA = 0.9

_NC = 2
_NS = 16
_NW = _NC * _NS
_K = 125
_NCHUNK = _E // _K
_CPT = _NCHUNK // _NS
_CHD = _NCHUNK // _NW
_RPS = 640
_TAIL = _N - 15 * _RPS
_ZR = 128

_RB = 1000


def _mesh():
    return plsc.VectorSubcoreMesh(core_axis_name="c", subcore_axis_name="s")


def _zero_slab(s, sp_ref, zbuf):
    lo = s * _RPS
    for t in range(_RPS // _ZR):
        @pl.when(jnp.logical_or(s < 15, t < _TAIL // _ZR))
        def _():
            pltpu.sync_copy(zbuf, sp_ref.at[pl.ds(lo + t * _ZR, _ZR)])

    @pl.when(s == 15)
    def _():
        pltpu.sync_copy(zbuf.at[pl.ds(0, _TAIL % _ZR)],
                        sp_ref.at[pl.ds(15 * _RPS + (_TAIL // _ZR) * _ZR,
                                        _TAIL % _ZR)])


def _dump(c, s, sp_ref, out):
    @pl.when(s < 15)
    def _():
        pltpu.sync_copy(sp_ref.at[pl.ds(s * _RPS, _RPS)],
                        out.at[c, pl.ds(s * _RPS, _RPS)])

    @pl.when(s == 15)
    def _():
        pltpu.sync_copy(sp_ref.at[pl.ds(15 * _RPS, _TAIL)],
                        out.at[c, pl.ds(15 * _RPS, _TAIL)])


def _deg_body(col2d, degp, col_v, ones_v, zbuf, deg_sp, ssa):
    c = lax.axis_index("c")
    s = lax.axis_index("s")
    w = c * _NS + s

    def fill_ones(i, carry):
        for t in range(_H // 16):
            ones_v[i, pl.ds(16 * t, 16)] = jnp.ones((16,), jnp.float32)
        return carry

    lax.fori_loop(0, _K, fill_ones, 0)

    def fill_z(i, carry):
        for t in range(_H // 16):
            zbuf[i, pl.ds(16 * t, 16)] = jnp.zeros((16,), jnp.float32)
        return carry

    lax.fori_loop(0, _ZR, fill_z, 0)

    _zero_slab(s, deg_sp, zbuf)
    pltpu.sync_copy(col2d.at[pl.ds(w * _CHD, _CHD)], col_v)
    plsc.subcore_barrier()

    def chunk(j, carry):
        pltpu.async_copy(ones_v, deg_sp.at[col_v.at[j]], ssa, add=True)

        @pl.when(j >= 8)
        def _():
            pltpu.make_async_copy(ones_v, deg_sp.at[col_v.at[j - 8]], ssa).wait()

        return carry

    lax.fori_loop(0, _CHD, chunk, 0)

    def drain(j, carry):
        pltpu.make_async_copy(ones_v, deg_sp.at[col_v.at[j]], ssa).wait()
        return carry

    lax.fori_loop(_CHD - 8, _CHD, drain, 0)
    plsc.subcore_barrier()
    _dump(c, s, deg_sp, degp)


_sc_deg = pl.kernel(
    _deg_body,
    out_type=jax.ShapeDtypeStruct((_NC, _N, _H), jnp.float32),
    mesh=_mesh(),
    scratch_types=[
        pltpu.VMEM((_CHD, _K), jnp.int32),
        pltpu.VMEM((_K, _H), jnp.float32),
        pltpu.VMEM((_ZR, _H), jnp.float32),
        pltpu.VMEM_SHARED((_N, _H), jnp.float32),
        pltpu.SemaphoreType.DMA,
    ],
    compiler_params=pltpu.CompilerParams(use_tc_tiling_on_sc=False),
    name="sc_deg_hist",
)


def _mp_body(h2, row2d, col2d, out, row_v, col_v, rba, rbb, zbuf, agg_sp,
             gsa, gsb, ssa, ssb):
    c = lax.axis_index("c")
    s = lax.axis_index("s")

    def fill_z(i, carry):
        for t in range(_H // 16):
            zbuf[i, pl.ds(16 * t, 16)] = jnp.zeros((16,), jnp.float32)
        return carry

    lax.fori_loop(0, _ZR, fill_z, 0)
    _zero_slab(s, agg_sp, zbuf)
    pltpu.sync_copy(row2d.at[pl.ds(s * _CPT, _CPT)], row_v)
    pltpu.sync_copy(col2d.at[pl.ds(s * _CPT, _CPT)], col_v)
    plsc.subcore_barrier()

    hb = h2.at[c]

    pltpu.async_copy(hb.at[row_v.at[0]], rba, gsa)

    def chunk(i, carry):
        j0 = 2 * i
        pltpu.make_async_copy(hb.at[row_v.at[j0]], rba, gsa).wait()

        @pl.when(i > 0)
        def _():
            pltpu.make_async_copy(rbb, agg_sp.at[col_v.at[j0 - 1]], ssb).wait()

        pltpu.async_copy(hb.at[row_v.at[j0 + 1]], rbb, gsb)
        pltpu.async_copy(rba, agg_sp.at[col_v.at[j0]], ssa, add=True)
        pltpu.make_async_copy(hb.at[row_v.at[j0 + 1]], rbb, gsb).wait()
        pltpu.make_async_copy(rba, agg_sp.at[col_v.at[j0]], ssa).wait()

        @pl.when(i < _CPT // 2 - 1)
        def _():
            pltpu.async_copy(hb.at[row_v.at[j0 + 2]], rba, gsa)

        pltpu.async_copy(rbb, agg_sp.at[col_v.at[j0 + 1]], ssb, add=True)
        return carry

    lax.fori_loop(0, _CPT // 2, chunk, 0)
    pltpu.make_async_copy(rbb, agg_sp.at[col_v.at[_CPT - 1]], ssb).wait()
    plsc.subcore_barrier()
    _dump(c, s, agg_sp, out)


_sc_mp = pl.kernel(
    _mp_body,
    out_type=jax.ShapeDtypeStruct((_NC, _N, _H), jnp.float32),
    mesh=_mesh(),
    scratch_types=[
        pltpu.VMEM((_CPT, _K), jnp.int32),
        pltpu.VMEM((_CPT, _K), jnp.int32),
        pltpu.VMEM((_K, _H), jnp.float32),
        pltpu.VMEM((_K, _H), jnp.float32),
        pltpu.VMEM((_ZR, _H), jnp.float32),
        pltpu.VMEM_SHARED((_N, _H), jnp.float32),
        pltpu.SemaphoreType.DMA,
        pltpu.SemaphoreType.DMA,
        pltpu.SemaphoreType.DMA,
        pltpu.SemaphoreType.DMA,
    ],
    compiler_params=pltpu.CompilerParams(use_tc_tiling_on_sc=False),
    name="sc_mp_round",
)


def _pairdot(v, w):
    f = w.shape[0]
    return jnp.concatenate(
        [jnp.dot(v[:, :f], w, preferred_element_type=jnp.float32),
         jnp.dot(v[:, f:], w, preferred_element_type=jnp.float32)], axis=1)


def _pre_body(dgp, x2, ds2, w0, b0, w11, b11, h128, h1128, hsb, dis):
    deg = dgp[0] + dgp[1]
    d = jnp.where(deg > 0, lax.rsqrt(jnp.maximum(deg, 1e-12)), 0.0)
    bb0 = jnp.concatenate([b0[...], b0[...]], axis=1)
    bb11 = jnp.concatenate([b11[...], b11[...]], axis=1)
    a = jnp.maximum(_pairdot(x2[...], w0[...]) + bb0, 0.0)
    b = jnp.maximum(_pairdot(ds2[...], w11[...]) + bb11, 0.0)
    h128[...] = a
    h1128[...] = b
    hsb[...] = d * jnp.stack([a, b], axis=0)
    dis[...] = d


_pre = pl.pallas_call(
    _pre_body,
    grid=(_N2 // _RB,),
    in_specs=[
        pl.BlockSpec((_NC, _RB, 128), lambda i: (0, i, 0)),
        pl.BlockSpec((_RB, 2 * _DF), lambda i: (i, 0)),
        pl.BlockSpec((_RB, 128), lambda i: (i, 0)),
        pl.BlockSpec((_DF, _H), lambda i: (0, 0)),
        pl.BlockSpec((1, _H), lambda i: (0, 0)),
        pl.BlockSpec((_H, _H), lambda i: (0, 0)),
        pl.BlockSpec((1, _H), lambda i: (0, 0)),
    ],
    out_specs=[
        pl.BlockSpec((_RB, 128), lambda i: (i, 0)),
        pl.BlockSpec((_RB, 128), lambda i: (i, 0)),
        pl.BlockSpec((_NC, _RB, 128), lambda i: (0, i, 0)),
        pl.BlockSpec((_RB, 128), lambda i: (i, 0)),
    ],
    out_shape=[
        jax.ShapeDtypeStruct((_N2, 128), jnp.float32),
        jax.ShapeDtypeStruct((_N2, 128), jnp.float32),
        jax.ShapeDtypeStruct((_NC, _N2, 128), jnp.float32),
        jax.ShapeDtypeStruct((_N2, 128), jnp.float32),
    ],
    name="tc_pre",
)


def _mix(beta, p, dis, h0, h10, w1a, w2a, w1b, w2b):
    d = dis[...]
    oa = _mix_half(beta, (1.0 - _ALPHA) * (d * p[0]), _ALPHA * h0[...], w1a, w2a)
    ob = _mix_half(beta, (1.0 - _ALPHA) * (d * p[1]), _ALPHA * h10[...], w1b, w2b)
    return oa, ob, d


def _mix_half(beta, aggh, h0a, w1, w2):
    o = (1.0 - beta) * aggh + beta * _pairdot(aggh, w1[...])
    o = o + (1.0 - beta) * h0a + beta * _pairdot(h0a, w2[...])
    return jnp.maximum(o, 0.0)


def _comb_mid_body(beta, p, dis, h0, h10, w1a, w2a, w1b, w2b, hsb):
    oa, ob, d = _mix(beta, p, dis, h0, h10, w1a, w2a, w1b, w2b)
    hsb[...] = d * jnp.stack([oa, ob], axis=0)


def _comb_last_body(beta, p, dis, h0, h10, w1a, w2a, w1b, w2b,
                    lwa, lba, lwb, lbb, z2):
    oa, ob, _ = _mix(beta, p, dis, h0, h10, w1a, w2a, w1b, w2b)
    za = _pairdot(oa, lwa[...]) + lba[...]
    zb = _pairdot(ob, lwb[...]) + lbb[...]
    z2[...] = jnp.stack([za, zb], axis=0)


_W_SPECS = [
    pl.BlockSpec((_NC, _RB, 128), lambda i: (0, i, 0)),
    pl.BlockSpec((_RB, 128), lambda i: (i, 0)),
    pl.BlockSpec((_RB, 128), lambda i: (i, 0)),
    pl.BlockSpec((_RB, 128), lambda i: (i, 0)),
    pl.BlockSpec((_H, _H), lambda i: (0, 0)),
    pl.BlockSpec((_H, _H), lambda i: (0, 0)),
    pl.BlockSpec((_H, _H), lambda i: (0, 0)),
    pl.BlockSpec((_H, _H), lambda i: (0, 0)),
]

_comb_mid0 = pl.pallas_call(
    functools.partial(_comb_mid_body, float(np.log(_THETA / 1.0 + 1.0))),
    grid=(_N2 // _RB,),
    in_specs=_W_SPECS,
    out_specs=[pl.BlockSpec((_NC, _RB, 128), lambda i: (0, i, 0))],
    out_shape=[jax.ShapeDtypeStruct((_NC, _N2, 128), jnp.float32)],
    name="tc_combine_mid",
)

_comb_last1 = pl.pallas_call(
    functools.partial(_comb_last_body, float(np.log(_THETA / 2.0 + 1.0))),
    grid=(_N2 // _RB,),
    in_specs=_W_SPECS + [
        pl.BlockSpec((_H, 1), lambda i: (0, 0)),
        pl.BlockSpec((1, 1), lambda i: (0, 0)),
        pl.BlockSpec((_H, 1), lambda i: (0, 0)),
        pl.BlockSpec((1, 1), lambda i: (0, 0)),
    ],
    out_specs=[pl.BlockSpec((_NC, _RB, 2), lambda i: (0, i, 0))],
    out_shape=[jax.ShapeDtypeStruct((_NC, _N2, 2), jnp.float32)],
    name="tc_combine_last",
)


def kernel(x, data_str, edge_index, lins0_w, lins0_b, lins1_w, lins1_b,
           lin11_w, lin11_b, lin3_w, lin3_b,
           convs_w1, convs_w2, convs1_w1, convs1_w2):
    row2d = edge_index[0].reshape(_NCHUNK, _K)
    col2d = edge_index[1].reshape(_NCHUNK, _K)

    x2 = x.reshape(_N2, 2 * _DF)
    ds64 = jnp.pad(data_str, ((0, 0), (0, _H - _DS)))
    ds2 = ds64.reshape(_N2, 128)
    w11p = jnp.pad(lin11_w, ((0, _H - _DS), (0, 0)))

    degp = _sc_deg(col2d)
    h, h1, hsb, dis = _pre(degp.reshape(_NC, _N2, 128), x2, ds2,
                           lins0_w, lins0_b.reshape(1, _H),
                           w11p, lin11_b.reshape(1, _H))

    p = _sc_mp(hsb.reshape(_NC, _N, _H), row2d, col2d)
    (hsb,) = _comb_mid0(p.reshape(_NC, _N2, 128), dis, h, h1,
                        convs_w1[0], convs_w2[0], convs1_w1[0], convs1_w2[0])
    p = _sc_mp(hsb.reshape(_NC, _N, _H), row2d, col2d)
    (z2,) = _comb_last1(p.reshape(_NC, _N2, 128), dis, h, h1,
                        convs_w1[1], convs_w2[1], convs1_w1[1], convs1_w2[1],
                        lins1_w, lins1_b.reshape(1, 1),
                        lin3_w, lin3_b.reshape(1, 1))
    z = z2[0].reshape(_N, 1)
    z1 = z2[1].reshape(_N, 1)
    return (z, z1)


# speedup vs baseline: 1.0275x; 1.0275x over previous
import functools

import numpy as np
import jax
import jax.numpy as jnp
from jax import lax
from jax.experimental import pallas as pl
from jax.experimental.pallas import tpu as pltpu
from jax.experimental.pallas import tpu_sc as plsc

_N = 10000
_N2 = _N // 2
_E = 320000
_DF = 128
_DS = 58
_H = 64
_ALPHA = 0.4
_THETA = 0.9

_NC = 2
_NS = 16
_NW = _NC * _NS
_K = 125
_NCHUNK = _E // _K
_CPT = _NCHUNK // _NS
_CHD = _NCHUNK // _NW
_RPS = 640
_TAIL = _N - 15 * _RPS
_ZR = 128

_RB = 1000


def _mesh():
    return plsc.VectorSubcoreMesh(core_axis_name="c", subcore_axis_name="s")


def _zero_slab(s, sp_ref, zbuf):
    lo = s * _RPS
    for t in range(_RPS // _ZR):
        @pl.when(jnp.logical_or(s < 15, t < _TAIL // _ZR))
        def _():
            pltpu.sync_copy(zbuf, sp_ref.at[pl.ds(lo + t * _ZR, _ZR)])

    @pl.when(s == 15)
    def _():
        pltpu.sync_copy(zbuf.at[pl.ds(0, _TAIL % _ZR)],
                        sp_ref.at[pl.ds(15 * _RPS + (_TAIL // _ZR) * _ZR,
                                        _TAIL % _ZR)])


def _dump(c, s, sp_ref, out):
    @pl.when(s < 15)
    def _():
        pltpu.sync_copy(sp_ref.at[pl.ds(s * _RPS, _RPS)],
                        out.at[c, pl.ds(s * _RPS, _RPS)])

    @pl.when(s == 15)
    def _():
        pltpu.sync_copy(sp_ref.at[pl.ds(15 * _RPS, _TAIL)],
                        out.at[c, pl.ds(15 * _RPS, _TAIL)])


def _deg_body(col2d, degp, col_v, ones_v, zbuf, deg_sp, ssa):
    c = lax.axis_index("c")
    s = lax.axis_index("s")
    w = c * _NS + s

    def fill_ones(i, carry):
        ones_v[i, :] = jnp.ones((16,), jnp.float32)
        return carry

    lax.fori_loop(0, _K, fill_ones, 0)

    def fill_z(i, carry):
        zbuf[i, :] = jnp.zeros((16,), jnp.float32)
        return carry

    lax.fori_loop(0, _ZR, fill_z, 0)

    _zero_slab(s, deg_sp, zbuf)
    pltpu.sync_copy(col2d.at[pl.ds(w * _CHD, _CHD)], col_v)
    plsc.subcore_barrier()

    def chunk(j, carry):
        pltpu.async_copy(ones_v, deg_sp.at[col_v.at[j]], ssa, add=True)

        @pl.when(j >= 8)
        def _():
            pltpu.make_async_copy(ones_v, deg_sp.at[col_v.at[j - 8]], ssa).wait()

        return carry

    lax.fori_loop(0, _CHD, chunk, 0)

    def drain(j, carry):
        pltpu.make_async_copy(ones_v, deg_sp.at[col_v.at[j]], ssa).wait()
        return carry

    lax.fori_loop(_CHD - 8, _CHD, drain, 0)
    plsc.subcore_barrier()
    _dump(c, s, deg_sp, degp)


_sc_deg = pl.kernel(
    _deg_body,
    out_type=jax.ShapeDtypeStruct((_NC, _N, 16), jnp.float32),
    mesh=_mesh(),
    scratch_types=[
        pltpu.VMEM((_CHD, _K), jnp.int32),
        pltpu.VMEM((_K, 16), jnp.float32),
        pltpu.VMEM((_ZR, 16), jnp.float32),
        pltpu.VMEM_SHARED((_N, 16), jnp.float32),
        pltpu.SemaphoreType.DMA,
    ],
    compiler_params=pltpu.CompilerParams(use_tc_tiling_on_sc=False),
    name="sc_deg_hist",
)


def _mp_body(h2, row2d, col2d, out, row_v, col_v, rba, rbb, zbuf, agg_sp,
             gsa, gsb, ssa, ssb):
    c = lax.axis_index("c")
    s = lax.axis_index("s")

    def fill_z(i, carry):
        for t in range(_H // 16):
            zbuf[i, pl.ds(16 * t, 16)] = jnp.zeros((16,), jnp.float32)
        return carry

    lax.fori_loop(0, _ZR, fill_z, 0)
    _zero_slab(s, agg_sp, zbuf)
    pltpu.sync_copy(row2d.at[pl.ds(s * _CPT, _CPT)], row_v)
    pltpu.sync_copy(col2d.at[pl.ds(s * _CPT, _CPT)], col_v)
    plsc.subcore_barrier()

    hb = h2.at[c]

    pltpu.async_copy(hb.at[row_v.at[0]], rba, gsa)

    def chunk(i, carry):
        j0 = 2 * i
        pltpu.make_async_copy(hb.at[row_v.at[j0]], rba, gsa).wait()

        @pl.when(i > 0)
        def _():
            pltpu.make_async_copy(rbb, agg_sp.at[col_v.at[j0 - 1]], ssb).wait()

        pltpu.async_copy(hb.at[row_v.at[j0 + 1]], rbb, gsb)
        pltpu.async_copy(rba, agg_sp.at[col_v.at[j0]], ssa, add=True)
        pltpu.make_async_copy(hb.at[row_v.at[j0 + 1]], rbb, gsb).wait()
        pltpu.make_async_copy(rba, agg_sp.at[col_v.at[j0]], ssa).wait()

        @pl.when(i < _CPT // 2 - 1)
        def _():
            pltpu.async_copy(hb.at[row_v.at[j0 + 2]], rba, gsa)

        pltpu.async_copy(rbb, agg_sp.at[col_v.at[j0 + 1]], ssb, add=True)
        return carry

    lax.fori_loop(0, _CPT // 2, chunk, 0)
    pltpu.make_async_copy(rbb, agg_sp.at[col_v.at[_CPT - 1]], ssb).wait()
    plsc.subcore_barrier()
    _dump(c, s, agg_sp, out)


_sc_mp = pl.kernel(
    _mp_body,
    out_type=jax.ShapeDtypeStruct((_NC, _N, _H), jnp.float32),
    mesh=_mesh(),
    scratch_types=[
        pltpu.VMEM((_CPT, _K), jnp.int32),
        pltpu.VMEM((_CPT, _K), jnp.int32),
        pltpu.VMEM((_K, _H), jnp.float32),
        pltpu.VMEM((_K, _H), jnp.float32),
        pltpu.VMEM((_ZR, _H), jnp.float32),
        pltpu.VMEM_SHARED((_N, _H), jnp.float32),
        pltpu.SemaphoreType.DMA,
        pltpu.SemaphoreType.DMA,
        pltpu.SemaphoreType.DMA,
        pltpu.SemaphoreType.DMA,
    ],
    compiler_params=pltpu.CompilerParams(use_tc_tiling_on_sc=False),
    name="sc_mp_round",
)


def _dis_body(dgp, dis):
    deg8 = dgp[0] + dgp[1]
    l = lax.broadcasted_iota(jnp.int32, (128, 128), 0)
    j = lax.broadcasted_iota(jnp.int32, (128, 128), 1)
    ys = []
    for k in range(4):
        mk = jnp.where((l == 32 * k) & (j < 64), 1.0,
                       jnp.where((l == 32 * k + 16) & (j >= 64), 1.0, 0.0))
        ys.append(jnp.dot(deg8, mk, preferred_element_type=jnp.float32))
    deg = jnp.stack(ys, axis=1).reshape(_N2, 128)
    dis[...] = jnp.where(deg > 0, lax.rsqrt(jnp.maximum(deg, 1e-12)), 0.0)


_dis = pl.pallas_call(
    _dis_body,
    out_shape=jax.ShapeDtypeStruct((_N2, 128), jnp.float32),
    name="tc_dis",
)


def _pre_body(dis_in, x2, ds2, w0b, b0b, w11b, b11b, h128, h1128, hsb):
    d = dis_in[...]
    a = jnp.maximum(
        jnp.dot(x2[...], w0b[...], preferred_element_type=jnp.float32) + b0b[...], 0.0)
    b = jnp.maximum(
        jnp.dot(ds2[...], w11b[...], preferred_element_type=jnp.float32) + b11b[...], 0.0)
    h128[...] = a
    h1128[...] = b
    hsb[...] = d * jnp.stack([a, b], axis=0)


_pre = pl.pallas_call(
    _pre_body,
    grid=(_N2 // _RB,),
    in_specs=[
        pl.BlockSpec((_RB, 128), lambda i: (i, 0)),
        pl.BlockSpec((_RB, 2 * _DF), lambda i: (i, 0)),
        pl.BlockSpec((_RB, 128), lambda i: (i, 0)),
        pl.BlockSpec((2 * _DF, 128), lambda i: (0, 0)),
        pl.BlockSpec((1, 128), lambda i: (0, 0)),
        pl.BlockSpec((128, 128), lambda i: (0, 0)),
        pl.BlockSpec((1, 128), lambda i: (0, 0)),
    ],
    out_specs=[
        pl.BlockSpec((_RB, 128), lambda i: (i, 0)),
        pl.BlockSpec((_RB, 128), lambda i: (i, 0)),
        pl.BlockSpec((_NC, _RB, 128), lambda i: (0, i, 0)),
    ],
    out_shape=[
        jax.ShapeDtypeStruct((_N2, 128), jnp.float32),
        jax.ShapeDtypeStruct((_N2, 128), jnp.float32),
        jax.ShapeDtypeStruct((_NC, _N2, 128), jnp.float32),
    ],
    name="tc_pre",
)


def _mix(beta, p, dis, h0, h10, w1a, w2a, w1b, w2b):
    d = dis[...]
    oa = _mix_half(beta, (1.0 - _ALPHA) * (d * p[0]), _ALPHA * h0[...], w1a, w2a)
    ob = _mix_half(beta, (1.0 - _ALPHA) * (d * p[1]), _ALPHA * h10[...], w1b, w2b)
    return oa, ob, d


def _mix_half(beta, aggh, h0a, w1, w2):
    o = (1.0 - beta) * aggh + beta * jnp.dot(
        aggh, w1[...], preferred_element_type=jnp.float32)
    o = o + (1.0 - beta) * h0a + beta * jnp.dot(
        h0a, w2[...], preferred_element_type=jnp.float32)
    return jnp.maximum(o, 0.0)


def _comb_mid_body(beta, p, dis, h0, h10, w1a, w2a, w1b, w2b, hsb):
    oa, ob, d = _mix(beta, p, dis, h0, h10, w1a, w2a, w1b, w2b)
    hsb[...] = d * jnp.stack([oa, ob], axis=0)


def _comb_last_body(beta, p, dis, h0, h10, w1a, w2a, w1b, w2b,
                    lwa, lwb, lb2, z2):
    oa, ob, _ = _mix(beta, p, dis, h0, h10, w1a, w2a, w1b, w2b)
    za = jnp.dot(oa, lwa[...], preferred_element_type=jnp.float32)
    zb = jnp.dot(ob, lwb[...], preferred_element_type=jnp.float32)
    z2[...] = jnp.stack([za, zb], axis=0) + lb2[...]


_W_SPECS = [
    pl.BlockSpec((_NC, _RB, 128), lambda i: (0, i, 0)),
    pl.BlockSpec((_RB, 128), lambda i: (i, 0)),
    pl.BlockSpec((_RB, 128), lambda i: (i, 0)),
    pl.BlockSpec((_RB, 128), lambda i: (i, 0)),
    pl.BlockSpec((128, 128), lambda i: (0, 0)),
    pl.BlockSpec((128, 128), lambda i: (0, 0)),
    pl.BlockSpec((128, 128), lambda i: (0, 0)),
    pl.BlockSpec((128, 128), lambda i: (0, 0)),
]

_comb_mid0 = pl.pallas_call(
    functools.partial(_comb_mid_body, float(np.log(_THETA / 1.0 + 1.0))),
    grid=(_N2 // _RB,),
    in_specs=_W_SPECS,
    out_specs=[pl.BlockSpec((_NC, _RB, 128), lambda i: (0, i, 0))],
    out_shape=[jax.ShapeDtypeStruct((_NC, _N2, 128), jnp.float32)],
    name="tc_combine_mid",
)

_comb_last1 = pl.pallas_call(
    functools.partial(_comb_last_body, float(np.log(_THETA / 2.0 + 1.0))),
    grid=(_N2 // _RB,),
    in_specs=_W_SPECS + [
        pl.BlockSpec((128, 2), lambda i: (0, 0)),
        pl.BlockSpec((128, 2), lambda i: (0, 0)),
        pl.BlockSpec((_NC, 1, 2), lambda i: (0, 0, 0)),
    ],
    out_specs=[pl.BlockSpec((_NC, _RB, 2), lambda i: (0, i, 0))],
    out_shape=[jax.ShapeDtypeStruct((_NC, _N2, 2), jnp.float32)],
    name="tc_combine_last",
)


def _blkdiag(w):
    fi, fo = w.shape
    zz = jnp.zeros((fi, fo), jnp.float32)
    return jnp.concatenate([
        jnp.concatenate([w, zz], axis=1),
        jnp.concatenate([zz, w], axis=1),
    ], axis=0)


def kernel(x, data_str, edge_index, lins0_w, lins0_b, lins1_w, lins1_b,
           lin11_w, lin11_b, lin3_w, lin3_b,
           convs_w1, convs_w2, convs1_w1, convs1_w2):
    row2d = edge_index[0].reshape(_NCHUNK, _K)
    col2d = edge_index[1].reshape(_NCHUNK, _K)

    x2 = x.reshape(_N2, 2 * _DF)
    ds64 = jnp.pad(data_str, ((0, 0), (0, _H - _DS)))
    ds2 = ds64.reshape(_N2, 128)
    w11p = jnp.pad(lin11_w, ((0, _H - _DS), (0, 0)))
    b2 = jnp.concatenate([lins0_b, lins0_b]).reshape(1, 128)
    b112 = jnp.concatenate([lin11_b, lin11_b]).reshape(1, 128)

    degp = _sc_deg(col2d)
    dis = _dis(degp.reshape(_NC, _N // 8, 128))
    h, h1, hsb = _pre(dis, x2, ds2,
                      _blkdiag(lins0_w), b2, _blkdiag(w11p), b112)

    p = _sc_mp(hsb.reshape(_NC, _N, _H), row2d, col2d)
    (hsb,) = _comb_mid0(p.reshape(_NC, _N2, 128), dis, h, h1,
                        _blkdiag(convs_w1[0]), _blkdiag(convs_w2[0]),
                        _blkdiag(convs1_w1[0]), _blkdiag(convs1_w2[0]))
    p = _sc_mp(hsb.reshape(_NC, _N, _H), row2d, col2d)
    lb2 = jnp.stack([jnp.broadcast_to(lins1_b, (2,)),
                     jnp.broadcast_to(lin3_b, (2,))]).reshape(_NC, 1, 2)
    (z2,) = _comb_last1(p.reshape(_NC, _N2, 128), dis, h, h1,
                        _blkdiag(convs_w1[1]), _blkdiag(convs_w2[1]),
                        _blkdiag(convs1_w1[1]), _blkdiag(convs1_w2[1]),
                        _blkdiag(lins1_w), _blkdiag(lin3_w), lb2)
    z = z2[0].reshape(_N, 1)
    z1 = z2[1].reshape(_N, 1)
    return (z, z1)


# speedup vs baseline: 1.0335x; 1.0059x over previous
import functools

import numpy as np
import jax
import jax.numpy as jnp
from jax import lax
from jax.experimental import pallas as pl
from jax.experimental.pallas import tpu as pltpu
from jax.experimental.pallas import tpu_sc as plsc

_N = 10000
_N2 = _N // 2
_E = 320000
_DF = 128
_DS = 58
_H = 64
_ALPHA = 0.4
_THETA = 0.9

_NC = 2
_NS = 16
_NW = _NC * _NS
_K = 125
_NCHUNK = _E // _K
_CPT = _NCHUNK // _NS
_CHD = _NCHUNK // _NW
_RPS = 640
_TAIL = _N - 15 * _RPS
_ZR = 128

_RB = 1000


def _mesh():
    return plsc.VectorSubcoreMesh(core_axis_name="c", subcore_axis_name="s")


def _zero_slab(s, sp_ref, zbuf):
    lo = s * _RPS
    for t in range(_RPS // _ZR):
        @pl.when(jnp.logical_or(s < 15, t < _TAIL // _ZR))
        def _():
            pltpu.sync_copy(zbuf, sp_ref.at[pl.ds(lo + t * _ZR, _ZR)])

    @pl.when(s == 15)
    def _():
        pltpu.sync_copy(zbuf.at[pl.ds(0, _TAIL % _ZR)],
                        sp_ref.at[pl.ds(15 * _RPS + (_TAIL // _ZR) * _ZR,
                                        _TAIL % _ZR)])


def _dump(c, s, sp_ref, out):
    @pl.when(s < 15)
    def _():
        pltpu.sync_copy(sp_ref.at[pl.ds(s * _RPS, _RPS)],
                        out.at[c, pl.ds(s * _RPS, _RPS)])

    @pl.when(s == 15)
    def _():
        pltpu.sync_copy(sp_ref.at[pl.ds(15 * _RPS, _TAIL)],
                        out.at[c, pl.ds(15 * _RPS, _TAIL)])


def _deg_body(col2d, degp, col_v, ones_v, zbuf, deg_sp, ssa):
    c = lax.axis_index("c")
    s = lax.axis_index("s")
    w = c * _NS + s

    def fill_ones(i, carry):
        ones_v[i, :] = jnp.ones((16,), jnp.float32)
        return carry

    lax.fori_loop(0, _K, fill_ones, 0)

    def fill_z(i, carry):
        zbuf[i, :] = jnp.zeros((16,), jnp.float32)
        return carry

    lax.fori_loop(0, _ZR, fill_z, 0)

    _zero_slab(s, deg_sp, zbuf)
    pltpu.sync_copy(col2d.at[pl.ds(w * _CHD, _CHD)], col_v)
    plsc.subcore_barrier()

    def chunk(j, carry):
        pltpu.async_copy(ones_v, deg_sp.at[col_v.at[j]], ssa, add=True)

        @pl.when(j >= 8)
        def _():
            pltpu.make_async_copy(ones_v, deg_sp.at[col_v.at[j - 8]], ssa).wait()

        return carry

    lax.fori_loop(0, _CHD, chunk, 0)

    def drain(j, carry):
        pltpu.make_async_copy(ones_v, deg_sp.at[col_v.at[j]], ssa).wait()
        return carry

    lax.fori_loop(_CHD - 8, _CHD, drain, 0)
    plsc.subcore_barrier()
    _dump(c, s, deg_sp, degp)


_sc_deg = pl.kernel(
    _deg_body,
    out_type=jax.ShapeDtypeStruct((_NC, _N, 16), jnp.float32),
    mesh=_mesh(),
    scratch_types=[
        pltpu.VMEM((_CHD, _K), jnp.int32),
        pltpu.VMEM((_K, 16), jnp.float32),
        pltpu.VMEM((_ZR, 16), jnp.float32),
        pltpu.VMEM_SHARED((_N, 16), jnp.float32),
        pltpu.SemaphoreType.DMA,
    ],
    compiler_params=pltpu.CompilerParams(use_tc_tiling_on_sc=False),
    name="sc_deg_hist",
)


def _mp_body(h2, row2d, col2d, out, row_v, col_v, rba, rbb, zbuf, agg_sp,
             gsa, gsb, ssa, ssb):
    c = lax.axis_index("c")
    s = lax.axis_index("s")

    def fill_z(i, carry):
        for t in range(_H // 16):
            zbuf[i, pl.ds(16 * t, 16)] = jnp.zeros((16,), jnp.float32)
        return carry

    lax.fori_loop(0, _ZR, fill_z, 0)
    _zero_slab(s, agg_sp, zbuf)
    pltpu.sync_copy(row2d.at[pl.ds(s * _CPT, _CPT)], row_v)
    pltpu.sync_copy(col2d.at[pl.ds(s * _CPT, _CPT)], col_v)
    plsc.subcore_barrier()

    hb = h2.at[c]

    pltpu.async_copy(hb.at[row_v.at[0]], rba, gsa)

    def chunk(i, carry):
        j0 = 2 * i
        pltpu.make_async_copy(hb.at[row_v.at[j0]], rba, gsa).wait()

        @pl.when(i > 0)
        def _():
            pltpu.make_async_copy(rbb, agg_sp.at[col_v.at[j0 - 1]], ssb).wait()

        pltpu.async_copy(hb.at[row_v.at[j0 + 1]], rbb, gsb)
        pltpu.async_copy(rba, agg_sp.at[col_v.at[j0]], ssa, add=True)
        pltpu.make_async_copy(hb.at[row_v.at[j0 + 1]], rbb, gsb).wait()
        pltpu.make_async_copy(rba, agg_sp.at[col_v.at[j0]], ssa).wait()

        @pl.when(i < _CPT // 2 - 1)
        def _():
            pltpu.async_copy(hb.at[row_v.at[j0 + 2]], rba, gsa)

        pltpu.async_copy(rbb, agg_sp.at[col_v.at[j0 + 1]], ssb, add=True)
        return carry

    lax.fori_loop(0, _CPT // 2, chunk, 0)
    pltpu.make_async_copy(rbb, agg_sp.at[col_v.at[_CPT - 1]], ssb).wait()
    plsc.subcore_barrier()
    _dump(c, s, agg_sp, out)


_sc_mp = pl.kernel(
    _mp_body,
    out_type=jax.ShapeDtypeStruct((_NC, _N, _H), jnp.float32),
    mesh=_mesh(),
    scratch_types=[
        pltpu.VMEM((_CPT, _K), jnp.int32),
        pltpu.VMEM((_CPT, _K), jnp.int32),
        pltpu.VMEM((_K, _H), jnp.float32),
        pltpu.VMEM((_K, _H), jnp.float32),
        pltpu.VMEM((_ZR, _H), jnp.float32),
        pltpu.VMEM_SHARED((_N, _H), jnp.float32),
        pltpu.SemaphoreType.DMA,
        pltpu.SemaphoreType.DMA,
        pltpu.SemaphoreType.DMA,
        pltpu.SemaphoreType.DMA,
    ],
    compiler_params=pltpu.CompilerParams(use_tc_tiling_on_sc=False),
    name="sc_mp_round",
)


def _pre_body(dgp, x2, ds2, w0b, b0b, w11b, b11b, h128, h1128, hsb, dis):
    deg8 = dgp[0] + dgp[1]
    l = lax.broadcasted_iota(jnp.int32, (128, 128), 0)
    j = lax.broadcasted_iota(jnp.int32, (128, 128), 1)
    ys = []
    for k in range(4):
        mk = jnp.where((l == 32 * k) & (j < 64), 1.0,
                       jnp.where((l == 32 * k + 16) & (j >= 64), 1.0, 0.0))
        ys.append(jnp.dot(deg8, mk, preferred_element_type=jnp.float32))
    deg = jnp.stack(ys, axis=1).reshape(_N2, 128)
    d = jnp.where(deg > 0, lax.rsqrt(jnp.maximum(deg, 1e-12)), 0.0)
    a = jnp.maximum(
        jnp.dot(x2[...], w0b[...], preferred_element_type=jnp.float32) + b0b[...], 0.0)
    b = jnp.maximum(
        jnp.dot(ds2[...], w11b[...], preferred_element_type=jnp.float32) + b11b[...], 0.0)
    h128[...] = a
    h1128[...] = b
    hsb[...] = d * jnp.stack([a, b], axis=0)
    dis[...] = d


_pre = pl.pallas_call(
    _pre_body,
    out_shape=[
        jax.ShapeDtypeStruct((_N2, 128), jnp.float32),
        jax.ShapeDtypeStruct((_N2, 128), jnp.float32),
        jax.ShapeDtypeStruct((_NC, _N2, 128), jnp.float32),
        jax.ShapeDtypeStruct((_N2, 128), jnp.float32),
    ],
    name="tc_pre",
)


def _mix(beta, p, dis, h0, h10, w1a, w2a, w1b, w2b):
    d = dis[...]
    oa = _mix_half(beta, (1.0 - _ALPHA) * (d * p[0]), _ALPHA * h0[...], w1a, w2a)
    ob = _mix_half(beta, (1.0 - _ALPHA) * (d * p[1]), _ALPHA * h10[...], w1b, w2b)
    return oa, ob, d


def _mix_half(beta, aggh, h0a, w1, w2):
    o = (1.0 - beta) * aggh + beta * jnp.dot(
        aggh, w1[...], preferred_element_type=jnp.float32)
    o = o + (1.0 - beta) * h0a + beta * jnp.dot(
        h0a, w2[...], preferred_element_type=jnp.float32)
    return jnp.maximum(o, 0.0)


def _comb_mid_body(beta, p, dis, h0, h10, w1a, w2a, w1b, w2b, hsb):
    oa, ob, d = _mix(beta, p, dis, h0, h10, w1a, w2a, w1b, w2b)
    hsb[...] = d * jnp.stack([oa, ob], axis=0)


def _comb_last_body(beta, p, dis, h0, h10, w1a, w2a, w1b, w2b,
                    lwa, lwb, lb2, z2):
    oa, ob, _ = _mix(beta, p, dis, h0, h10, w1a, w2a, w1b, w2b)
    za = jnp.dot(oa, lwa[...], preferred_element_type=jnp.float32)
    zb = jnp.dot(ob, lwb[...], preferred_element_type=jnp.float32)
    z2[...] = jnp.stack([za, zb], axis=0) + lb2[...]


_W_SPECS = [
    pl.BlockSpec((_NC, _RB, 128), lambda i: (0, i, 0)),
    pl.BlockSpec((_RB, 128), lambda i: (i, 0)),
    pl.BlockSpec((_RB, 128), lambda i: (i, 0)),
    pl.BlockSpec((_RB, 128), lambda i: (i, 0)),
    pl.BlockSpec((128, 128), lambda i: (0, 0)),
    pl.BlockSpec((128, 128), lambda i: (0, 0)),
    pl.BlockSpec((128, 128), lambda i: (0, 0)),
    pl.BlockSpec((128, 128), lambda i: (0, 0)),
]

_comb_mid0 = pl.pallas_call(
    functools.partial(_comb_mid_body, float(np.log(_THETA / 1.0 + 1.0))),
    grid=(_N2 // _RB,),
    in_specs=_W_SPECS,
    out_specs=[pl.BlockSpec((_NC, _RB, 128), lambda i: (0, i, 0))],
    out_shape=[jax.ShapeDtypeStruct((_NC, _N2, 128), jnp.float32)],
    name="tc_combine_mid",
)

_comb_last1 = pl.pallas_call(
    functools.partial(_comb_last_body, float(np.log(_THETA / 2.0 + 1.0))),
    grid=(_N2 // _RB,),
    in_specs=_W_SPECS + [
        pl.BlockSpec((128, 2), lambda i: (0, 0)),
        pl.BlockSpec((128, 2), lambda i: (0, 0)),
        pl.BlockSpec((_NC, 1, 2), lambda i: (0, 0, 0)),
    ],
    out_specs=[pl.BlockSpec((_NC, _RB, 2), lambda i: (0, i, 0))],
    out_shape=[jax.ShapeDtypeStruct((_NC, _N2, 2), jnp.float32)],
    name="tc_combine_last",
)


def _blkdiag(w):
    fi, fo = w.shape
    zz = jnp.zeros((fi, fo), jnp.float32)
    return jnp.concatenate([
        jnp.concatenate([w, zz], axis=1),
        jnp.concatenate([zz, w], axis=1),
    ], axis=0)


def kernel(x, data_str, edge_index, lins0_w, lins0_b, lins1_w, lins1_b,
           lin11_w, lin11_b, lin3_w, lin3_b,
           convs_w1, convs_w2, convs1_w1, convs1_w2):
    row2d = edge_index[0].reshape(_NCHUNK, _K)
    col2d = edge_index[1].reshape(_NCHUNK, _K)

    x2 = x.reshape(_N2, 2 * _DF)
    ds64 = jnp.pad(data_str, ((0, 0), (0, _H - _DS)))
    ds2 = ds64.reshape(_N2, 128)
    w11p = jnp.pad(lin11_w, ((0, _H - _DS), (0, 0)))
    b2 = jnp.concatenate([lins0_b, lins0_b]).reshape(1, 128)
    b112 = jnp.concatenate([lin11_b, lin11_b]).reshape(1, 128)

    degp = _sc_deg(col2d)
    h, h1, hsb, dis = _pre(degp.reshape(_NC, _N // 8, 128), x2, ds2,
                           _blkdiag(lins0_w), b2, _blkdiag(w11p), b112)

    p = _sc_mp(hsb.reshape(_NC, _N, _H), row2d, col2d)
    (hsb,) = _comb_mid0(p.reshape(_NC, _N2, 128), dis, h, h1,
                        _blkdiag(convs_w1[0]), _blkdiag(convs_w2[0]),
                        _blkdiag(convs1_w1[0]), _blkdiag(convs1_w2[0]))
    p = _sc_mp(hsb.reshape(_NC, _N, _H), row2d, col2d)
    lb2 = jnp.stack([jnp.broadcast_to(lins1_b, (2,)),
                     jnp.broadcast_to(lin3_b, (2,))]).reshape(_NC, 1, 2)
    (z2,) = _comb_last1(p.reshape(_NC, _N2, 128), dis, h, h1,
                        _blkdiag(convs_w1[1]), _blkdiag(convs_w2[1]),
                        _blkdiag(convs1_w1[1]), _blkdiag(convs1_w2[1]),
                        _blkdiag(lins1_w), _blkdiag(lin3_w), lb2)
    z = z2[0].reshape(_N, 1)
    z1 = z2[1].reshape(_N, 1)
    return (z, z1)


# speedup vs baseline: 1.0344x; 1.0008x over previous
import functools

import numpy as np
import jax
import jax.numpy as jnp
from jax import lax
from jax.experimental import pallas as pl
from jax.experimental.pallas import tpu as pltpu
from jax.experimental.pallas import tpu_sc as plsc

_N = 10000
_N2 = _N // 2
_E = 320000
_DF = 128
_DS = 58
_H = 64
_ALPHA = 0.4
_THETA = 0.9

_NC = 2
_NS = 16
_NW = _NC * _NS
_K = 125
_NCHUNK = _E // _K
_CPT = _NCHUNK // _NS
_CHD = _NCHUNK // _NW
_RPS = 640
_TAIL = _N - 15 * _RPS
_ZR = 128

_RB = 1000


def _mesh():
    return plsc.VectorSubcoreMesh(core_axis_name="c", subcore_axis_name="s",
                                  num_cores=_NC, num_subcores=_NS)


def _zero_slab(s, sp_ref, zbuf):
    lo = s * _RPS
    for t in range(_RPS // _ZR):
        @pl.when(jnp.logical_or(s < 15, t < _TAIL // _ZR))
        def _():
            pltpu.sync_copy(zbuf, sp_ref.at[pl.ds(lo + t * _ZR, _ZR)])

    @pl.when(s == 15)
    def _():
        pltpu.sync_copy(zbuf.at[pl.ds(0, _TAIL % _ZR)],
                        sp_ref.at[pl.ds(15 * _RPS + (_TAIL // _ZR) * _ZR,
                                        _TAIL % _ZR)])


def _dump(c, s, sp_ref, out):
    @pl.when(s < 15)
    def _():
        pltpu.sync_copy(sp_ref.at[pl.ds(s * _RPS, _RPS)],
                        out.at[c, pl.ds(s * _RPS, _RPS)])

    @pl.when(s == 15)
    def _():
        pltpu.sync_copy(sp_ref.at[pl.ds(15 * _RPS, _TAIL)],
                        out.at[c, pl.ds(15 * _RPS, _TAIL)])


def _deg_body(col2d, degp, col_v, ones_v, zbuf, deg_sp, ssa):
    c = lax.axis_index("c")
    s = lax.axis_index("s")
    w = c * _NS + s

    def fill_ones(i, carry):
        ones_v[i, :] = jnp.ones((16,), jnp.float32)
        return carry

    lax.fori_loop(0, _K, fill_ones, 0)

    def fill_z(i, carry):
        zbuf[i, :] = jnp.zeros((16,), jnp.float32)
        return carry

    lax.fori_loop(0, _ZR, fill_z, 0)

    _zero_slab(s, deg_sp, zbuf)
    pltpu.sync_copy(col2d.at[pl.ds(w * _CHD, _CHD)], col_v)
    plsc.subcore_barrier()

    def chunk(j, carry):
        pltpu.async_copy(ones_v, deg_sp.at[col_v.at[j]], ssa, add=True)

        @pl.when(j >= 8)
        def _():
            pltpu.make_async_copy(ones_v, deg_sp.at[col_v.at[j - 8]], ssa).wait()

        return carry

    lax.fori_loop(0, _CHD, chunk, 0)

    def drain(j, carry):
        pltpu.make_async_copy(ones_v, deg_sp.at[col_v.at[j]], ssa).wait()
        return carry

    lax.fori_loop(_CHD - 8, _CHD, drain, 0)
    plsc.subcore_barrier()
    _dump(c, s, deg_sp, degp)


_sc_deg = pl.kernel(
    _deg_body,
    out_type=jax.ShapeDtypeStruct((_NC, _N, 16), jnp.float32),
    mesh=_mesh(),
    scratch_types=[
        pltpu.VMEM((_CHD, _K), jnp.int32),
        pltpu.VMEM((_K, 16), jnp.float32),
        pltpu.VMEM((_ZR, 16), jnp.float32),
        pltpu.VMEM_SHARED((_N, 16), jnp.float32),
        pltpu.SemaphoreType.DMA,
    ],
    compiler_params=pltpu.CompilerParams(use_tc_tiling_on_sc=False),
    name="sc_deg_hist",
)


def _mp_body(h2, row2d, col2d, out, row_v, col_v, rba, rbb, zbuf, agg_sp,
             gsa, gsb, ssa, ssb):
    c = lax.axis_index("c")
    s = lax.axis_index("s")

    def fill_z(i, carry):
        for t in range(_H // 16):
            zbuf[i, pl.ds(16 * t, 16)] = jnp.zeros((16,), jnp.float32)
        return carry

    lax.fori_loop(0, _ZR, fill_z, 0)
    _zero_slab(s, agg_sp, zbuf)
    pltpu.sync_copy(row2d.at[pl.ds(s * _CPT, _CPT)], row_v)
    pltpu.sync_copy(col2d.at[pl.ds(s * _CPT, _CPT)], col_v)
    plsc.subcore_barrier()

    hb = h2.at[c]

    pltpu.async_copy(hb.at[row_v.at[0]], rba, gsa)

    def chunk(i, carry):
        j0 = 2 * i
        pltpu.make_async_copy(hb.at[row_v.at[j0]], rba, gsa).wait()

        @pl.when(i > 0)
        def _():
            pltpu.make_async_copy(rbb, agg_sp.at[col_v.at[j0 - 1]], ssb).wait()

        pltpu.async_copy(hb.at[row_v.at[j0 + 1]], rbb, gsb)
        pltpu.async_copy(rba, agg_sp.at[col_v.at[j0]], ssa, add=True)
        pltpu.make_async_copy(hb.at[row_v.at[j0 + 1]], rbb, gsb).wait()
        pltpu.make_async_copy(rba, agg_sp.at[col_v.at[j0]], ssa).wait()

        @pl.when(i < _CPT // 2 - 1)
        def _():
            pltpu.async_copy(hb.at[row_v.at[j0 + 2]], rba, gsa)

        pltpu.async_copy(rbb, agg_sp.at[col_v.at[j0 + 1]], ssb, add=True)
        return carry

    lax.fori_loop(0, _CPT // 2, chunk, 0)
    pltpu.make_async_copy(rbb, agg_sp.at[col_v.at[_CPT - 1]], ssb).wait()
    plsc.subcore_barrier()
    _dump(c, s, agg_sp, out)


_sc_mp = pl.kernel(
    _mp_body,
    out_type=jax.ShapeDtypeStruct((_NC, _N, _H), jnp.float32),
    mesh=_mesh(),
    scratch_types=[
        pltpu.VMEM((_CPT, _K), jnp.int32),
        pltpu.VMEM((_CPT, _K), jnp.int32),
        pltpu.VMEM((_K, _H), jnp.float32),
        pltpu.VMEM((_K, _H), jnp.float32),
        pltpu.VMEM((_ZR, _H), jnp.float32),
        pltpu.VMEM_SHARED((_N, _H), jnp.float32),
        pltpu.SemaphoreType.DMA,
        pltpu.SemaphoreType.DMA,
        pltpu.SemaphoreType.DMA,
        pltpu.SemaphoreType.DMA,
    ],
    compiler_params=pltpu.CompilerParams(use_tc_tiling_on_sc=False),
    name="sc_mp_round",
)


def _pre_body(dgp, x2, ds2, w0b, b0b, w11b, b11b, h128, h1128, hsb, dis):
    deg8 = dgp[0] + dgp[1]
    l = lax.broadcasted_iota(jnp.int32, (128, 128), 0)
    j = lax.broadcasted_iota(jnp.int32, (128, 128), 1)
    ys = []
    for k in range(4):
        mk = jnp.where((l == 32 * k) & (j < 64), 1.0,
                       jnp.where((l == 32 * k + 16) & (j >= 64), 1.0, 0.0))
        ys.append(jnp.dot(deg8, mk, preferred_element_type=jnp.float32))
    deg = jnp.stack(ys, axis=1).reshape(_N2, 128)
    d = jnp.where(deg > 0, lax.rsqrt(jnp.maximum(deg, 1e-12)), 0.0)
    a = jnp.maximum(
        jnp.dot(x2[...], w0b[...], preferred_element_type=jnp.float32) + b0b[...], 0.0)
    b = jnp.maximum(
        jnp.dot(ds2[...], w11b[...], preferred_element_type=jnp.float32) + b11b[...], 0.0)
    h128[...] = a
    h1128[...] = b
    hsb[...] = d * jnp.stack([a, b], axis=0)
    dis[...] = d


_pre = pl.pallas_call(
    _pre_body,
    out_shape=[
        jax.ShapeDtypeStruct((_N2, 128), jnp.float32),
        jax.ShapeDtypeStruct((_N2, 128), jnp.float32),
        jax.ShapeDtypeStruct((_NC, _N2, 128), jnp.float32),
        jax.ShapeDtypeStruct((_N2, 128), jnp.float32),
    ],
    name="tc_pre",
)


def _mix(beta, p, dis, h0, h10, w1a, w2a, w1b, w2b):
    d = dis[...]
    oa = _mix_half(beta, (1.0 - _ALPHA) * (d * p[0]), _ALPHA * h0[...], w1a, w2a)
    ob = _mix_half(beta, (1.0 - _ALPHA) * (d * p[1]), _ALPHA * h10[...], w1b, w2b)
    return oa, ob, d


def _mix_half(beta, aggh, h0a, w1, w2):
    o = (1.0 - beta) * aggh + beta * jnp.dot(
        aggh, w1[...], preferred_element_type=jnp.float32)
    o = o + (1.0 - beta) * h0a + beta * jnp.dot(
        h0a, w2[...], preferred_element_type=jnp.float32)
    return jnp.maximum(o, 0.0)


def _comb_mid_body(beta, p, dis, h0, h10, w1a, w2a, w1b, w2b, hsb):
    oa, ob, d = _mix(beta, p, dis, h0, h10, w1a, w2a, w1b, w2b)
    hsb[...] = d * jnp.stack([oa, ob], axis=0)


def _comb_last_body(beta, p, dis, h0, h10, w1a, w2a, w1b, w2b,
                    lwa, lwb, lb2, z2):
    oa, ob, _ = _mix(beta, p, dis, h0, h10, w1a, w2a, w1b, w2b)
    za = jnp.dot(oa, lwa[...], preferred_element_type=jnp.float32)
    zb = jnp.dot(ob, lwb[...], preferred_element_type=jnp.float32)
    z2[...] = jnp.stack([za, zb], axis=0) + lb2[...]


_W_SPECS = [
    pl.BlockSpec((_NC, _RB, 128), lambda i: (0, i, 0)),
    pl.BlockSpec((_RB, 128), lambda i: (i, 0)),
    pl.BlockSpec((_RB, 128), lambda i: (i, 0)),
    pl.BlockSpec((_RB, 128), lambda i: (i, 0)),
    pl.BlockSpec((128, 128), lambda i: (0, 0)),
    pl.BlockSpec((128, 128), lambda i: (0, 0)),
    pl.BlockSpec((128, 128), lambda i: (0, 0)),
    pl.BlockSpec((128, 128), lambda i: (0, 0)),
]

_comb_mid0 = pl.pallas_call(
    functools.partial(_comb_mid_body, float(np.log(_THETA / 1.0 + 1.0))),
    grid=(_N2 // _RB,),
    in_specs=_W_SPECS,
    out_specs=[pl.BlockSpec((_NC, _RB, 128), lambda i: (0, i, 0))],
    out_shape=[jax.ShapeDtypeStruct((_NC, _N2, 128), jnp.float32)],
    name="tc_combine_mid",
)

_comb_last1 = pl.pallas_call(
    functools.partial(_comb_last_body, float(np.log(_THETA / 2.0 + 1.0))),
    grid=(_N2 // _RB,),
    in_specs=_W_SPECS + [
        pl.BlockSpec((128, 2), lambda i: (0, 0)),
        pl.BlockSpec((128, 2), lambda i: (0, 0)),
        pl.BlockSpec((_NC, 1, 2), lambda i: (0, 0, 0)),
    ],
    out_specs=[pl.BlockSpec((_NC, _RB, 2), lambda i: (0, i, 0))],
    out_shape=[jax.ShapeDtypeStruct((_NC, _N2, 2), jnp.float32)],
    name="tc_combine_last",
)


def _blkdiag(w):
    fi, fo = w.shape
    zz = jnp.zeros((fi, fo), jnp.float32)
    return jnp.concatenate([
        jnp.concatenate([w, zz], axis=1),
        jnp.concatenate([zz, w], axis=1),
    ], axis=0)


def kernel(x, data_str, edge_index, lins0_w, lins0_b, lins1_w, lins1_b,
           lin11_w, lin11_b, lin3_w, lin3_b,
           convs_w1, convs_w2, convs1_w1, convs1_w2):
    row2d = edge_index[0].reshape(_NCHUNK, _K)
    col2d = edge_index[1].reshape(_NCHUNK, _K)

    x2 = x.reshape(_N2, 2 * _DF)
    ds64 = jnp.pad(data_str, ((0, 0), (0, _H - _DS)))
    ds2 = ds64.reshape(_N2, 128)
    w11p = jnp.pad(lin11_w, ((0, _H - _DS), (0, 0)))
    b2 = jnp.concatenate([lins0_b, lins0_b]).reshape(1, 128)
    b112 = jnp.concatenate([lin11_b, lin11_b]).reshape(1, 128)

    degp = _sc_deg(col2d)
    h, h1, hsb, dis = _pre(degp.reshape(_NC, _N // 8, 128), x2, ds2,
                           _blkdiag(lins0_w), b2, _blkdiag(w11p), b112)

    p = _sc_mp(hsb.reshape(_NC, _N, _H), row2d, col2d)
    (hsb,) = _comb_mid0(p.reshape(_NC, _N2, 128), dis, h, h1,
                        _blkdiag(convs_w1[0]), _blkdiag(convs_w2[0]),
                        _blkdiag(convs1_w1[0]), _blkdiag(convs1_w2[0]))
    p = _sc_mp(hsb.reshape(_NC, _N, _H), row2d, col2d)
    lb2 = jnp.stack([jnp.broadcast_to(lins1_b, (2,)),
                     jnp.broadcast_to(lin3_b, (2,))]).reshape(_NC, 1, 2)
    (z2,) = _comb_last1(p.reshape(_NC, _N2, 128), dis, h, h1,
                        _blkdiag(convs_w1[1]), _blkdiag(convs_w2[1]),
                        _blkdiag(convs1_w1[1]), _blkdiag(convs1_w2[1]),
                        _blkdiag(lins1_w), _blkdiag(lin3_w), lb2)
    z = z2[0].reshape(_N, 1)
    z1 = z2[1].reshape(_N, 1)
    return (z, z1)
